# baseline (speedup 1.0000x reference)
import functools

import jax
import jax.numpy as jnp
from jax import lax
from jax.experimental import pallas as pl
from jax.experimental.pallas import tpu as pltpu

B = 8192
D = 4096
K = 8
CAP = 2048
H = 64
KH = K * H

_BM = 1024
_BK = 1024
_NK = D // _BK


_NST = B // _BM


def _score_body(ctx_hbm, w1_ref, b1_ref, w2_ref, b2_ref, out_ref, buf, sem):
    i = pl.program_id(0)
    slot = lax.rem(i, 2)
    nxt = lax.rem(i + 1, 2)

    @pl.when(i == 0)
    def _prime():
        pltpu.make_async_copy(
            ctx_hbm.at[pl.ds(0, _BM), :], buf.at[0], sem.at[0]).start()

    @pl.when(i + 1 < _NST)
    def _prefetch():
        pltpu.make_async_copy(
            ctx_hbm.at[pl.ds((i + 1) * _BM, _BM), :], buf.at[nxt], sem.at[nxt]).start()

    pltpu.make_async_copy(
        ctx_hbm.at[pl.ds(i * _BM, _BM), :], buf.at[slot], sem.at[slot]).wait()

    ctx = buf[slot]
    h = jnp.dot(ctx, w1_ref[...], preferred_element_type=jnp.float32)
    h = jax.nn.relu(h + b1_ref[...])
    s = jnp.dot(h, w2_ref[...], preferred_element_type=jnp.float32)
    out_ref[...] = s + b2_ref[...]


def _select_body(s_ref, a_ref, spm_ref, mps_ref, lbv_ref):
    s = s_ref[...]
    i = lax.bitcast_convert_type(s, jnp.int32)
    o = jnp.where(i >= 0, i, i ^ jnp.int32(0x7FFFFFFF))

    def tbody(it, t):
        bit = jnp.int32(30) - it
        cand = t + jnp.left_shift(jnp.int32(1), bit)
        cnt = jnp.sum((o >= cand).astype(jnp.int32), axis=1, keepdims=True)
        return jnp.where(cnt >= CAP, cand, t)

    cnt_pos = jnp.sum((o >= 0).astype(jnp.int32), axis=1, keepdims=True)
    t0 = jnp.where(cnt_pos >= CAP, jnp.int32(0),
                   jnp.full((K, 1), jnp.iinfo(jnp.int32).min, dtype=jnp.int32))
    T = lax.fori_loop(0, 31, tbody, t0)

    gt = o > T
    cnt_gt = jnp.sum(gt.astype(jnp.int32), axis=1, keepdims=True)
    r = jnp.int32(CAP) - cnt_gt
    eq = o == T
    idx = lax.broadcasted_iota(jnp.int32, (K, B), 1)

    def cbody(_, lohi):
        lo, hi = lohi
        mid = (lo + hi) >> 1
        cnt = jnp.sum((eq & (idx <= mid)).astype(jnp.int32), axis=1, keepdims=True)
        pred = cnt >= r
        return jnp.where(pred, lo, mid + 1), jnp.where(pred, mid, hi)

    lo0 = jnp.zeros((K, 1), jnp.int32)
    hi0 = jnp.full((K, 1), B - 1, jnp.int32)
    lo, hi = lax.fori_loop(0, 13, cbody, (lo0, hi0))

    sel = gt | (eq & (idx <= lo))
    m = jnp.max(s, axis=1, keepdims=True)
    e = jnp.where(sel, jnp.exp(s - m), 0.0)
    z = jnp.sum(e, axis=1, keepdims=True)
    a = e / z
    a_ref[...] = a

    spm = jnp.sum(a, axis=1, keepdims=True)
    spm_ref[...] = spm
    total = jnp.sum(spm)
    mps_ref[...] = jnp.full((1, 1), total / B, dtype=jnp.float32)
    mean = total / K
    dvar = spm - mean
    lbv_ref[...] = jnp.full((1, 1), jnp.sum(dvar * dvar) / (K - 1), dtype=jnp.float32)


try:
    from jax.experimental.pallas import tpu_sc as plsc
    _HAS_SC = True
except ImportError:
    _HAS_SC = False

_NCH = B // 16


def _sc_select_body(sT_hbm, oT_hbm, aT_hbm, spm_hbm,
                    sbuf, obuf, ebuf, hist, sfx, wbuf):
    cid = lax.axis_index("c")
    sid = lax.axis_index("s")
    lane = lax.iota(jnp.int32, 16)

    def ou_chunk(i):
        return obuf[pl.ds(i * 16, 16)]

    @pl.when((cid == 0) & (sid < K))
    def _work():
        k = sid
        pltpu.sync_copy(sT_hbm.at[k], sbuf)
        pltpu.sync_copy(oT_hbm.at[k], obuf)

        need = jnp.int32(CAP)
        prefix = jnp.uint32(0)
        m16 = jnp.full((16,), -jnp.inf, jnp.float32)
        eq_total = jnp.int32(0)
        ones = jnp.ones((16,), jnp.int32)

        for l in range(4):
            shift = 24 - 8 * l

            def _clr(j, c):
                hist[pl.ds(j * 16, 16)] = jnp.zeros((16,), jnp.int32)
                return c
            lax.fori_loop(0, 272, _clr, 0)

            if l == 0:
                def _h0(i, mcar):
                    s = sbuf[pl.ds(i * 16, 16)]
                    ou = ou_chunk(i)
                    band = jnp.right_shift(ou, jnp.uint32(shift)).astype(jnp.int32)
                    plsc.addupdate_scatter(hist, [band * 16 + lane], ones)
                    return jnp.maximum(mcar, s)
                m16 = lax.fori_loop(0, _NCH, _h0, m16)
            else:
                pref = prefix

                def _hl(i, c):
                    ou = ou_chunk(i)
                    ing = jnp.right_shift(ou, jnp.uint32(shift + 8)) == pref
                    band = jnp.right_shift(ou, jnp.uint32(shift)).astype(jnp.int32) & 255
                    fi = jnp.where(ing, band * 16 + lane, 4096 + lane)
                    plsc.addupdate_scatter(hist, [fi], ones)
                    return c
                lax.fori_loop(0, _NCH, _hl, 0)

            sfx[pl.ds(256 * 16, 16)] = jnp.zeros((16,), jnp.int32)

            def _sf(t, run):
                bb = 255 - t
                run = run + hist[pl.ds(bb * 16, 16)]
                sfx[pl.ds(bb * 16, 16)] = run
                return run
            lax.fori_loop(0, 256, _sf, jnp.zeros((16,), jnp.int32))

            nd = need

            def _bs(t, b):
                cand = b + jnp.left_shift(jnp.int32(1), 7 - t)
                cnt = jnp.sum(sfx[pl.ds(cand * 16, 16)])
                return jnp.where(cnt >= nd, cand, b)
            bstar = lax.fori_loop(0, 8, _bs, jnp.int32(0))

            above = jnp.sum(sfx[pl.ds((bstar + 1) * 16, 16)])
            if l == 3:
                eq_total = jnp.sum(sfx[pl.ds(bstar * 16, 16)]) - above
            need = need - above
            prefix = jnp.left_shift(prefix, jnp.uint32(8)) | bstar.astype(jnp.uint32)

        T_ou = prefix
        r = need

        def _serial():
            def _sc(i, car):
                cnt, cut = car
                ou = ou_chunk(i)
                eq = ou == T_ou
                pr = lax.cumsum(eq.astype(jnp.int32), axis=0)
                hit = eq & ((cnt + pr) <= r)
                cidx = jnp.where(hit, i * 16 + lane, jnp.int32(-1))
                return cnt + jnp.sum(eq.astype(jnp.int32)), jnp.maximum(cut, jnp.max(cidx))
            return lax.fori_loop(0, _NCH, _sc, (jnp.int32(0), jnp.int32(-1)))[1]

        cut = lax.cond(eq_total == r, lambda: jnp.int32(B - 1), _serial)
        mscal = jnp.max(m16)

        def _sel(i, z16):
            s = sbuf[pl.ds(i * 16, 16)]
            ou = ou_chunk(i)
            idxv = i * 16 + lane
            sel = (ou > T_ou) | ((ou == T_ou) & (idxv <= cut))
            e = jnp.where(sel, jnp.exp(s - mscal), jnp.float32(0.0))
            ebuf[pl.ds(i * 16, 16)] = e
            return z16 + e
        z16 = lax.fori_loop(0, _NCH, _sel, jnp.zeros((16,), jnp.float32))
        inv_z = jnp.ones((16,), jnp.float32) / jnp.full((16,), jnp.sum(z16), jnp.float32)

        def _out(i, a16):
            a = ebuf[pl.ds(i * 16, 16)] * inv_z
            sbuf[pl.ds(i * 16, 16)] = a
            return a16 + a
        a16 = lax.fori_loop(0, _NCH, _out, jnp.zeros((16,), jnp.float32))
        pltpu.sync_copy(sbuf, aT_hbm.at[k])

        wbuf[...] = jnp.full((16,), jnp.sum(a16), jnp.float32)
        pltpu.sync_copy(wbuf, spm_hbm.at[k])


def _sc_select(scores_T):
    u = lax.bitcast_convert_type(scores_T, jnp.uint32)
    msb = jnp.uint32(0x80000000)
    scores_T_ou = jnp.where(u >= msb, ~u, u | msb)

    mesh = plsc.VectorSubcoreMesh(core_axis_name="c", subcore_axis_name="s")
    fn = pl.kernel(
        _sc_select_body,
        mesh=mesh,
        out_type=(
            jax.ShapeDtypeStruct((K, B), jnp.float32),
            jax.ShapeDtypeStruct((K, 16), jnp.float32),
        ),
        scratch_types=[
            pltpu.VMEM((B,), jnp.float32),
            pltpu.VMEM((B,), jnp.uint32),
            pltpu.VMEM((B,), jnp.float32),
            pltpu.VMEM((4352,), jnp.int32),
            pltpu.VMEM((4112,), jnp.int32),
            pltpu.VMEM((16,), jnp.float32),
        ],
        compiler_params=pltpu.CompilerParams(needs_layout_passes=False),
    )
    return fn(scores_T, scores_T_ou)


@jax.jit
def kernel(context, W1, b1, W2, b2):
    W1r = W1.transpose(1, 0, 2).reshape(D, KH)
    b1r = b1.reshape(1, KH)
    W2b = (W2[:, :, None] * jnp.eye(K, dtype=W2.dtype)[:, None, :]).reshape(KH, K)
    b2r = b2.reshape(1, K)

    scores = pl.pallas_call(
        _score_body,
        grid=(B // _BM,),
        in_specs=[
            pl.BlockSpec(memory_space=pl.ANY),
            pl.BlockSpec((D, KH), lambda i: (0, 0)),
            pl.BlockSpec((1, KH), lambda i: (0, 0)),
            pl.BlockSpec((KH, K), lambda i: (0, 0)),
            pl.BlockSpec((1, K), lambda i: (0, 0)),
        ],
        out_specs=pl.BlockSpec((_BM, K), lambda i: (i, 0)),
        out_shape=jax.ShapeDtypeStruct((B, K), jnp.float32),
        scratch_shapes=[
            pltpu.VMEM((2, _BM, D), jnp.float32),
            pltpu.SemaphoreType.DMA((2,)),
        ],
    )(context, W1r, b1r, W2b, b2r)

    scores_T = scores.T

    a_T, spm16 = _sc_select(scores_T)

    assignment = a_T.T
    spm = spm16[:, 0]
    total = jnp.sum(spm)
    mps = total / B
    lbv = jnp.sum((spm - total / K) ** 2) / (K - 1)
    return assignment, scores, spm, mps, lbv


# speedup vs baseline: 1.2606x; 1.2606x over previous
import functools

import jax
import jax.numpy as jnp
from jax import lax
from jax.experimental import pallas as pl
from jax.experimental.pallas import tpu as pltpu

B = 8192
D = 4096
K = 8
CAP = 2048
H = 64
KH = K * H

_BM = 1024
_BK = 1024
_NK = D // _BK


_NST = B // _BM


def _score_body(ctx_hbm, w1_ref, b1_ref, w2_ref, b2_ref, out_ref, buf, sem):
    i = pl.program_id(0)
    slot = lax.rem(i, 2)
    nxt = lax.rem(i + 1, 2)

    @pl.when(i == 0)
    def _prime():
        pltpu.make_async_copy(
            ctx_hbm.at[pl.ds(0, _BM), :], buf.at[0], sem.at[0]).start()

    @pl.when(i + 1 < _NST)
    def _prefetch():
        pltpu.make_async_copy(
            ctx_hbm.at[pl.ds((i + 1) * _BM, _BM), :], buf.at[nxt], sem.at[nxt]).start()

    pltpu.make_async_copy(
        ctx_hbm.at[pl.ds(i * _BM, _BM), :], buf.at[slot], sem.at[slot]).wait()

    ctx = buf[slot]
    h = jnp.dot(ctx, w1_ref[...], preferred_element_type=jnp.float32)
    h = jax.nn.relu(h + b1_ref[...])
    s = jnp.dot(h, w2_ref[...], preferred_element_type=jnp.float32)
    out_ref[...] = s + b2_ref[...]


def _select_body(s_ref, a_ref, spm_ref, mps_ref, lbv_ref):
    s = s_ref[...]
    i = lax.bitcast_convert_type(s, jnp.int32)
    o = jnp.where(i >= 0, i, i ^ jnp.int32(0x7FFFFFFF))

    def tbody(it, t):
        bit = jnp.int32(30) - it
        cand = t + jnp.left_shift(jnp.int32(1), bit)
        cnt = jnp.sum((o >= cand).astype(jnp.int32), axis=1, keepdims=True)
        return jnp.where(cnt >= CAP, cand, t)

    cnt_pos = jnp.sum((o >= 0).astype(jnp.int32), axis=1, keepdims=True)
    t0 = jnp.where(cnt_pos >= CAP, jnp.int32(0),
                   jnp.full((K, 1), jnp.iinfo(jnp.int32).min, dtype=jnp.int32))
    T = lax.fori_loop(0, 31, tbody, t0)

    gt = o > T
    cnt_gt = jnp.sum(gt.astype(jnp.int32), axis=1, keepdims=True)
    r = jnp.int32(CAP) - cnt_gt
    eq = o == T
    idx = lax.broadcasted_iota(jnp.int32, (K, B), 1)

    def cbody(_, lohi):
        lo, hi = lohi
        mid = (lo + hi) >> 1
        cnt = jnp.sum((eq & (idx <= mid)).astype(jnp.int32), axis=1, keepdims=True)
        pred = cnt >= r
        return jnp.where(pred, lo, mid + 1), jnp.where(pred, mid, hi)

    lo0 = jnp.zeros((K, 1), jnp.int32)
    hi0 = jnp.full((K, 1), B - 1, jnp.int32)
    lo, hi = lax.fori_loop(0, 13, cbody, (lo0, hi0))

    sel = gt | (eq & (idx <= lo))
    m = jnp.max(s, axis=1, keepdims=True)
    e = jnp.where(sel, jnp.exp(s - m), 0.0)
    z = jnp.sum(e, axis=1, keepdims=True)
    a = e / z
    a_ref[...] = a

    spm = jnp.sum(a, axis=1, keepdims=True)
    spm_ref[...] = spm
    total = jnp.sum(spm)
    mps_ref[...] = jnp.full((1, 1), total / B, dtype=jnp.float32)
    mean = total / K
    dvar = spm - mean
    lbv_ref[...] = jnp.full((1, 1), jnp.sum(dvar * dvar) / (K - 1), dtype=jnp.float32)


try:
    from jax.experimental.pallas import tpu_sc as plsc
    _HAS_SC = True
except ImportError:
    _HAS_SC = False

_NCH = B // 16


def _sc_select_body(sT_hbm, oT_hbm, aT_hbm, spm_hbm,
                    sbuf, obuf, ebuf, hist, sfx, wbuf):
    cid = lax.axis_index("c")
    sid = lax.axis_index("s")
    lane = lax.iota(jnp.int32, 16)

    def ou_chunk(i):
        return obuf[pl.ds(i * 16, 16)]

    @pl.when((cid == 0) & (sid < K))
    def _work():
        k = sid
        pltpu.sync_copy(sT_hbm.at[k], sbuf)
        pltpu.sync_copy(oT_hbm.at[k], obuf)

        need = jnp.int32(CAP)
        prefix = jnp.uint32(0)
        m16 = jnp.full((16,), -jnp.inf, jnp.float32)
        eq_total = jnp.int32(0)
        ones = jnp.ones((16,), jnp.int32)

        for l in range(4):
            shift = 24 - 8 * l

            @functools.partial(plsc.parallel_loop, 0, 272, unroll=8)
            def _clr(j):
                hist[pl.ds(j * 16, 16)] = jnp.zeros((16,), jnp.int32)

            if l == 0:
                def _h0(i, mcar):
                    s = sbuf[pl.ds(i * 16, 16)]
                    ou = ou_chunk(i)
                    band = jnp.right_shift(ou, jnp.uint32(shift)).astype(jnp.int32)
                    plsc.addupdate_scatter(hist, [band * 16 + lane], ones)
                    return jnp.maximum(mcar, s)
                m16 = plsc.parallel_loop(0, _NCH, unroll=8, carry=m16)(_h0)
            else:
                pref = prefix

                def _hl(i, c):
                    ou = ou_chunk(i)
                    ing = jnp.right_shift(ou, jnp.uint32(shift + 8)) == pref
                    band = jnp.right_shift(ou, jnp.uint32(shift)).astype(jnp.int32) & 255
                    fi = jnp.where(ing, band * 16 + lane, 4096 + lane)
                    plsc.addupdate_scatter(hist, [fi], ones)
                    return c
                plsc.parallel_loop(0, _NCH, unroll=8, carry=jnp.int32(0))(_hl)

            sfx[pl.ds(256 * 16, 16)] = jnp.zeros((16,), jnp.int32)

            def _sf(t, run):
                bb = 255 - t
                run = run + hist[pl.ds(bb * 16, 16)]
                sfx[pl.ds(bb * 16, 16)] = run
                return run
            plsc.parallel_loop(0, 256, unroll=8,
                               carry=jnp.zeros((16,), jnp.int32))(_sf)

            nd = need

            def _bs(t, b):
                cand = b + jnp.left_shift(jnp.int32(1), 7 - t)
                cnt = jnp.sum(sfx[pl.ds(cand * 16, 16)])
                return jnp.where(cnt >= nd, cand, b)
            bstar = lax.fori_loop(0, 8, _bs, jnp.int32(0))

            above = jnp.sum(sfx[pl.ds((bstar + 1) * 16, 16)])
            if l == 3:
                eq_total = jnp.sum(sfx[pl.ds(bstar * 16, 16)]) - above
            need = need - above
            prefix = jnp.left_shift(prefix, jnp.uint32(8)) | bstar.astype(jnp.uint32)

        T_ou = prefix
        r = need

        def _serial():
            def _sc(i, car):
                cnt, cut = car
                ou = ou_chunk(i)
                eq = ou == T_ou
                pr = lax.cumsum(eq.astype(jnp.int32), axis=0)
                hit = eq & ((cnt + pr) <= r)
                cidx = jnp.where(hit, i * 16 + lane, jnp.int32(-1))
                return cnt + jnp.sum(eq.astype(jnp.int32)), jnp.maximum(cut, jnp.max(cidx))
            return lax.fori_loop(0, _NCH, _sc, (jnp.int32(0), jnp.int32(-1)))[1]

        cut = lax.cond(eq_total == r, lambda: jnp.int32(B - 1), _serial)
        mscal = jnp.max(m16)

        def _sel(i, z16):
            s = sbuf[pl.ds(i * 16, 16)]
            ou = ou_chunk(i)
            idxv = i * 16 + lane
            sel = (ou > T_ou) | ((ou == T_ou) & (idxv <= cut))
            e = jnp.where(sel, jnp.exp(s - mscal), jnp.float32(0.0))
            ebuf[pl.ds(i * 16, 16)] = e
            return z16 + e
        z16 = plsc.parallel_loop(0, _NCH, unroll=8,
                                 carry=jnp.zeros((16,), jnp.float32))(_sel)
        inv_z = jnp.ones((16,), jnp.float32) / jnp.full((16,), jnp.sum(z16), jnp.float32)

        def _out(i, a16):
            a = ebuf[pl.ds(i * 16, 16)] * inv_z
            sbuf[pl.ds(i * 16, 16)] = a
            return a16 + a
        a16 = plsc.parallel_loop(0, _NCH, unroll=8,
                                 carry=jnp.zeros((16,), jnp.float32))(_out)
        pltpu.sync_copy(sbuf, aT_hbm.at[k])

        wbuf[...] = jnp.full((16,), jnp.sum(a16), jnp.float32)
        pltpu.sync_copy(wbuf, spm_hbm.at[k])


def _sc_select(scores_T):
    u = lax.bitcast_convert_type(scores_T, jnp.uint32)
    msb = jnp.uint32(0x80000000)
    scores_T_ou = jnp.where(u >= msb, ~u, u | msb)

    mesh = plsc.VectorSubcoreMesh(core_axis_name="c", subcore_axis_name="s")
    fn = pl.kernel(
        _sc_select_body,
        mesh=mesh,
        out_type=(
            jax.ShapeDtypeStruct((K, B), jnp.float32),
            jax.ShapeDtypeStruct((K, 16), jnp.float32),
        ),
        scratch_types=[
            pltpu.VMEM((B,), jnp.float32),
            pltpu.VMEM((B,), jnp.uint32),
            pltpu.VMEM((B,), jnp.float32),
            pltpu.VMEM((4352,), jnp.int32),
            pltpu.VMEM((4112,), jnp.int32),
            pltpu.VMEM((16,), jnp.float32),
        ],
        compiler_params=pltpu.CompilerParams(needs_layout_passes=False),
    )
    return fn(scores_T, scores_T_ou)


@jax.jit
def kernel(context, W1, b1, W2, b2):
    W1r = W1.transpose(1, 0, 2).reshape(D, KH)
    b1r = b1.reshape(1, KH)
    W2b = (W2[:, :, None] * jnp.eye(K, dtype=W2.dtype)[:, None, :]).reshape(KH, K)
    b2r = b2.reshape(1, K)

    scores = pl.pallas_call(
        _score_body,
        grid=(B // _BM,),
        in_specs=[
            pl.BlockSpec(memory_space=pl.ANY),
            pl.BlockSpec((D, KH), lambda i: (0, 0)),
            pl.BlockSpec((1, KH), lambda i: (0, 0)),
            pl.BlockSpec((KH, K), lambda i: (0, 0)),
            pl.BlockSpec((1, K), lambda i: (0, 0)),
        ],
        out_specs=pl.BlockSpec((_BM, K), lambda i: (i, 0)),
        out_shape=jax.ShapeDtypeStruct((B, K), jnp.float32),
        scratch_shapes=[
            pltpu.VMEM((2, _BM, D), jnp.float32),
            pltpu.SemaphoreType.DMA((2,)),
        ],
    )(context, W1r, b1r, W2b, b2r)

    scores_T = scores.T

    a_T, spm16 = _sc_select(scores_T)

    assignment = a_T.T
    spm = spm16[:, 0]
    total = jnp.sum(spm)
    mps = total / B
    lbv = jnp.sum((spm - total / K) ** 2) / (K - 1)
    return assignment, scores, spm, mps, lbv
